# Initial kernel scaffold; baseline (speedup 1.0000x reference)
import jax
import jax.numpy as jnp
from jax.experimental import pallas as pl


def kernel(input_ids, attention_mask, wte, wpe):
    raise NotImplementedError("write your pallas kernel here")


# speedup vs baseline: 1.0045x; 1.0045x over previous
import jax
import jax.numpy as jnp
from jax import lax
from jax.experimental import pallas as pl
from jax.experimental.pallas import tpu as pltpu
from jax.experimental.pallas import tpu_sc as plsc

_VOCAB = 100000
_MAX_POS = 8192
_D = 1024
_B = 4
_S = 8192

_NC = 2
_NS = 16
_NW = _NC * _NS
_N = _B * _S
_PW = _N // _NW
_C = 32
_NCHUNK = _PW // _C
_LANES = 16


def _body(ids_hbm, mask_hbm, wte_hbm, wpe_hbm, out_hbm, am_hbm,
          idx_v, buf_a, buf_b, mbuf, sem):
    wid = lax.axis_index("s") * _NC + lax.axis_index("c")
    base = wid * _PW
    pos0 = lax.rem(base, _S)

    pltpu.sync_copy(mask_hbm.at[pl.ds(base, _PW)], mbuf)

    def mask_step(i, carry):
        sl = pl.ds(i * _LANES, _LANES)
        mbuf[sl] = (1.0 - mbuf[sl]) * -10000.0
        return carry

    lax.fori_loop(0, _PW // _LANES, mask_step, 0)
    pltpu.sync_copy(mbuf, am_hbm.at[pl.ds(base, _PW)])

    pltpu.sync_copy(ids_hbm.at[pl.ds(base, _PW)], idx_v)

    def chunk_step(c, carry):
        off = c * _C
        cp_wte = pltpu.async_copy(wte_hbm.at[idx_v.at[pl.ds(off, _C)]], buf_a, sem)
        cp_wpe = pltpu.async_copy(wpe_hbm.at[pl.ds(pos0 + off, _C)], buf_b, sem)
        cp_wte.wait()
        cp_wpe.wait()

        @plsc.parallel_loop(0, _C * _D, step=_LANES, unroll=8)
        def add_body(i):
            r = i >> 10
            sl = pl.ds(pl.multiple_of(i & (_D - 1), _LANES), _LANES)
            buf_a[r, sl] = buf_a[r, sl] + buf_b[r, sl]

        pltpu.sync_copy(buf_a, out_hbm.at[pl.ds(base + off, _C)])
        return carry

    lax.fori_loop(0, _NCHUNK, chunk_step, 0)


def _make_kernel():
    mesh = plsc.VectorSubcoreMesh(core_axis_name="c", subcore_axis_name="s")
    return pl.kernel(
        _body,
        out_type=(
            jax.ShapeDtypeStruct((_N, _D), jnp.float32),
            jax.ShapeDtypeStruct((_N,), jnp.float32),
        ),
        mesh=mesh,
        scratch_types=[
            pltpu.VMEM((_PW,), jnp.int32),
            pltpu.VMEM((_C, _D), jnp.float32),
            pltpu.VMEM((_C, _D), jnp.float32),
            pltpu.VMEM((_PW,), jnp.float32),
            pltpu.SemaphoreType.DMA,
        ],
    )


def kernel(input_ids, attention_mask, wte, wpe):
    b, s = input_ids.shape
    ids = input_ids.reshape(-1).astype(jnp.int32)
    maskf = attention_mask.astype(jnp.float32).reshape(-1)
    out, am = _make_kernel()(ids, maskf, wte, wpe)
    return out.reshape(b, s, _D), am.reshape(b, 1, 1, s)


# speedup vs baseline: 1.5858x; 1.5787x over previous
import jax
import jax.numpy as jnp
from jax import lax
from jax.experimental import pallas as pl
from jax.experimental.pallas import tpu as pltpu
from jax.experimental.pallas import tpu_sc as plsc

_D = 1024
_B = 4
_S = 8192

_NC = 2
_NS = 16
_NW = _NC * _NS
_N = _B * _S
_PP = _S // _NW
_C = 32
_NCHUNK = _PP // _C
_NSTEP = _NCHUNK * _B
_LANES = 16


def _body(ids_hbm, mask_hbm, wte_hbm, wpe_hbm, out_hbm, am_hbm,
          idx_v, a0, a1, wbuf, mbuf, sem_g, sem_o, sem_w):
    wid = lax.axis_index("s") * _NC + lax.axis_index("c")
    pos0 = wid * _PP
    abuf = (a0, a1)

    for b in range(_B):
        pltpu.sync_copy(ids_hbm.at[pl.ds(b * _S + pos0, _PP)],
                        idx_v.at[pl.ds(b * _PP, _PP)])

    def gather(t):
        c, b = divmod(t, _B)
        return pltpu.async_copy(
            wte_hbm.at[idx_v.at[pl.ds(b * _PP + c * _C, _C)]],
            abuf[b % 2], sem_g)

    def store(t):
        c, b = divmod(t, _B)
        return pltpu.async_copy(
            abuf[b % 2], out_hbm.at[pl.ds(b * _S + pos0 + c * _C, _C)],
            sem_o)

    def load_wpe(c):
        return pltpu.async_copy(
            wpe_hbm.at[pl.ds(pos0 + c * _C, _C)], wbuf, sem_w)

    cp_w = load_wpe(0)
    g_next = gather(0)

    mbase = wid * (_N // _NW)
    pltpu.sync_copy(mask_hbm.at[pl.ds(mbase, _N // _NW)], mbuf)

    @plsc.parallel_loop(0, _N // _NW, step=_LANES, unroll=4)
    def mask_step(i):
        sl = pl.ds(pl.multiple_of(i, _LANES), _LANES)
        mbuf[sl] = (1.0 - mbuf[sl]) * -10000.0

    pltpu.sync_copy(mbuf, am_hbm.at[pl.ds(mbase, _N // _NW)])

    s_prev = None
    for t in range(_NSTEP):
        c, b = divmod(t, _B)
        if b == 0:
            cp_w.wait()
        g_next.wait()
        if s_prev is not None:
            s_prev.wait()
        if t + 1 < _NSTEP:
            g_next = gather(t + 1)

        buf = abuf[b % 2]

        @plsc.parallel_loop(0, _C * _D, step=_LANES, unroll=8)
        def add_body(i):
            r = i >> 10
            sl = pl.ds(pl.multiple_of(i & (_D - 1), _LANES), _LANES)
            buf[r, sl] = buf[r, sl] + wbuf[r, sl]

        if b == _B - 1 and c + 1 < _NCHUNK:
            cp_w = load_wpe(c + 1)
        s_prev = store(t)

    s_prev.wait()


def _make_kernel():
    mesh = plsc.VectorSubcoreMesh(core_axis_name="c", subcore_axis_name="s")
    return pl.kernel(
        _body,
        out_type=(
            jax.ShapeDtypeStruct((_N, _D), jnp.float32),
            jax.ShapeDtypeStruct((_N,), jnp.float32),
        ),
        mesh=mesh,
        scratch_types=[
            pltpu.VMEM((_B * _PP,), jnp.int32),
            pltpu.VMEM((_C, _D), jnp.float32),
            pltpu.VMEM((_C, _D), jnp.float32),
            pltpu.VMEM((_C, _D), jnp.float32),
            pltpu.VMEM((_N // _NW,), jnp.float32),
            pltpu.SemaphoreType.DMA,
            pltpu.SemaphoreType.DMA,
            pltpu.SemaphoreType.DMA,
        ],
    )


def kernel(input_ids, attention_mask, wte, wpe):
    b, s = input_ids.shape
    ids = input_ids.reshape(-1).astype(jnp.int32)
    maskf = attention_mask.astype(jnp.float32).reshape(-1)
    out, am = _make_kernel()(ids, maskf, wte, wpe)
    return out.reshape(b, s, _D), am.reshape(b, 1, 1, s)


# speedup vs baseline: 1.8187x; 1.1469x over previous
import jax
import jax.numpy as jnp
from jax import lax
from jax.experimental import pallas as pl
from jax.experimental.pallas import tpu as pltpu
from jax.experimental.pallas import tpu_sc as plsc

_D = 1024
_B = 4
_S = 8192

_NC = 2
_NS = 16
_NW = _NC * _NS
_N = _B * _S
_PP = _S // _NW
_G = 8
_NSTEP = _PP // _G
_ROWS = _B * _G
_LANES = 16


def _body(ids_hbm, mask_hbm, wte_hbm, wpe_hbm, out_hbm, am_hbm,
          idx_v, a0, a1, w0, w1, mbuf, sem_g, sem_o, sem_w):
    wid = lax.axis_index("s") * _NC + lax.axis_index("c")
    pos0 = wid * _PP
    abuf = (a0, a1)
    wbuf = (w0, w1)

    for b in range(_B):
        pltpu.sync_copy(ids_hbm.at[pl.ds(b * _S + pos0, _PP)],
                        idx_v.at[pl.ds(b * _PP, _PP)])

    def gather(t):
        buf = abuf[t % 2]
        return [
            pltpu.async_copy(
                wte_hbm.at[idx_v.at[pl.ds(b * _PP + t * _G, _G)]],
                buf.at[pl.ds(b * _G, _G)], sem_g)
            for b in range(_B)
        ]

    def store(t):
        buf = abuf[t % 2]
        return [
            pltpu.async_copy(
                buf.at[pl.ds(b * _G, _G)],
                out_hbm.at[pl.ds(b * _S + pos0 + t * _G, _G)], sem_o)
            for b in range(_B)
        ]

    def load_wpe(t):
        return pltpu.async_copy(
            wpe_hbm.at[pl.ds(pos0 + t * _G, _G)], wbuf[t % 2], sem_w)

    cp_w = [load_wpe(0), load_wpe(1)]
    g_next = gather(0)

    mbase = wid * (_N // _NW)
    pltpu.sync_copy(mask_hbm.at[pl.ds(mbase, _N // _NW)], mbuf)

    @plsc.parallel_loop(0, _N // _NW, step=_LANES, unroll=4)
    def mask_step(i):
        sl = pl.ds(pl.multiple_of(i, _LANES), _LANES)
        mbuf[sl] = (1.0 - mbuf[sl]) * -10000.0

    pltpu.sync_copy(mbuf, am_hbm.at[pl.ds(mbase, _N // _NW)])

    s_prev = None
    for t in range(_NSTEP):
        cp_w[t % 2].wait()
        for g in g_next:
            g.wait()
        if s_prev is not None:
            for s in s_prev:
                s.wait()
        if t + 1 < _NSTEP:
            g_next = gather(t + 1)

        buf = abuf[t % 2]
        wb = wbuf[t % 2]

        @plsc.parallel_loop(0, _G * _D, step=_LANES, unroll=4)
        def add_body(i):
            p = i >> 10
            sl = pl.ds(pl.multiple_of(i & (_D - 1), _LANES), _LANES)
            wv = wb[p, sl]
            for b in range(_B):
                buf[b * _G + p, sl] = buf[b * _G + p, sl] + wv

        if t + 2 < _NSTEP:
            cp_w[t % 2] = load_wpe(t + 2)
        s_prev = store(t)

    for s in s_prev:
        s.wait()


def _make_kernel():
    mesh = plsc.VectorSubcoreMesh(core_axis_name="c", subcore_axis_name="s")
    return pl.kernel(
        _body,
        out_type=(
            jax.ShapeDtypeStruct((_N, _D), jnp.float32),
            jax.ShapeDtypeStruct((_N,), jnp.float32),
        ),
        mesh=mesh,
        scratch_types=[
            pltpu.VMEM((_B * _PP,), jnp.int32),
            pltpu.VMEM((_ROWS, _D), jnp.float32),
            pltpu.VMEM((_ROWS, _D), jnp.float32),
            pltpu.VMEM((_G, _D), jnp.float32),
            pltpu.VMEM((_G, _D), jnp.float32),
            pltpu.VMEM((_N // _NW,), jnp.float32),
            pltpu.SemaphoreType.DMA,
            pltpu.SemaphoreType.DMA,
            pltpu.SemaphoreType.DMA,
        ],
    )


def kernel(input_ids, attention_mask, wte, wpe):
    b, s = input_ids.shape
    ids = input_ids.reshape(-1).astype(jnp.int32)
    maskf = attention_mask.astype(jnp.float32).reshape(-1)
    out, am = _make_kernel()(ids, maskf, wte, wpe)
    return out.reshape(b, s, _D), am.reshape(b, 1, 1, s)


# speedup vs baseline: 1.8235x; 1.0026x over previous
import jax
import jax.numpy as jnp
from jax import lax
from jax.experimental import pallas as pl
from jax.experimental.pallas import tpu as pltpu
from jax.experimental.pallas import tpu_sc as plsc

_D = 1024
_B = 4
_S = 8192

_NC = 2
_NS = 16
_NW = _NC * _NS
_N = _B * _S
_PP = _S // _NW
_G = 8
_NSTEP = _PP // _G
_ROWS = _B * _G
_LANES = 16


def _body(ids_hbm, mask_hbm, wte_hbm, wpe_hbm, out_hbm, am_hbm,
          idx_v, a0, a1, a2, w0, w1, mbuf, sem_g, sem_o, sem_w):
    wid = lax.axis_index("s") * _NC + lax.axis_index("c")
    pos0 = wid * _PP
    abuf = (a0, a1, a2)
    wbuf = (w0, w1)

    for b in range(_B):
        pltpu.sync_copy(ids_hbm.at[pl.ds(b * _S + pos0, _PP)],
                        idx_v.at[pl.ds(b * _PP, _PP)])

    def gather(t):
        buf = abuf[t % 3]
        return [
            pltpu.async_copy(
                wte_hbm.at[idx_v.at[pl.ds(b * _PP + t * _G, _G)]],
                buf.at[pl.ds(b * _G, _G)], sem_g)
            for b in range(_B)
        ]

    def store(t):
        buf = abuf[t % 3]
        return [
            pltpu.async_copy(
                buf.at[pl.ds(b * _G, _G)],
                out_hbm.at[pl.ds(b * _S + pos0 + t * _G, _G)], sem_o)
            for b in range(_B)
        ]

    def load_wpe(t):
        return pltpu.async_copy(
            wpe_hbm.at[pl.ds(pos0 + t * _G, _G)], wbuf[t % 2], sem_w)

    cp_w = [load_wpe(0), load_wpe(1)]
    g_next = gather(0)

    mbase = wid * (_N // _NW)
    pltpu.sync_copy(mask_hbm.at[pl.ds(mbase, _N // _NW)], mbuf)

    @plsc.parallel_loop(0, _N // _NW, step=_LANES, unroll=4)
    def mask_step(i):
        sl = pl.ds(pl.multiple_of(i, _LANES), _LANES)
        mbuf[sl] = (1.0 - mbuf[sl]) * -10000.0

    pltpu.sync_copy(mbuf, am_hbm.at[pl.ds(mbase, _N // _NW)])

    stores = [None] * _NSTEP
    for t in range(_NSTEP):
        cp_w[t % 2].wait()
        for g in g_next:
            g.wait()
        if t >= 2:
            for s in stores[t - 2]:
                s.wait()
        if t + 1 < _NSTEP:
            g_next = gather(t + 1)

        buf = abuf[t % 3]
        wb = wbuf[t % 2]

        @plsc.parallel_loop(0, _G * _D, step=_LANES, unroll=4)
        def add_body(i):
            p = i >> 10
            sl = pl.ds(pl.multiple_of(i & (_D - 1), _LANES), _LANES)
            wv = wb[p, sl]
            for b in range(_B):
                buf[b * _G + p, sl] = buf[b * _G + p, sl] + wv

        if t + 2 < _NSTEP:
            cp_w[t % 2] = load_wpe(t + 2)
        stores[t] = store(t)

    for t in (_NSTEP - 2, _NSTEP - 1):
        for s in stores[t]:
            s.wait()


def _make_kernel():
    mesh = plsc.VectorSubcoreMesh(core_axis_name="c", subcore_axis_name="s")
    return pl.kernel(
        _body,
        out_type=(
            jax.ShapeDtypeStruct((_N, _D), jnp.float32),
            jax.ShapeDtypeStruct((_N,), jnp.float32),
        ),
        mesh=mesh,
        scratch_types=[
            pltpu.VMEM((_B * _PP,), jnp.int32),
            pltpu.VMEM((_ROWS, _D), jnp.float32),
            pltpu.VMEM((_ROWS, _D), jnp.float32),
            pltpu.VMEM((_ROWS, _D), jnp.float32),
            pltpu.VMEM((_G, _D), jnp.float32),
            pltpu.VMEM((_G, _D), jnp.float32),
            pltpu.VMEM((_N // _NW,), jnp.float32),
            pltpu.SemaphoreType.DMA,
            pltpu.SemaphoreType.DMA,
            pltpu.SemaphoreType.DMA,
        ],
    )


def kernel(input_ids, attention_mask, wte, wpe):
    b, s = input_ids.shape
    ids = input_ids.reshape(-1).astype(jnp.int32)
    maskf = attention_mask.astype(jnp.float32).reshape(-1)
    out, am = _make_kernel()(ids, maskf, wte, wpe)
    return out.reshape(b, s, _D), am.reshape(b, 1, 1, s)
